# Initial kernel scaffold; baseline (speedup 1.0000x reference)
import jax
import jax.numpy as jnp
from jax.experimental import pallas as pl


def kernel(x, edge_index, W):
    raise NotImplementedError("write your pallas kernel here")


# speedup vs baseline: 3.0879x; 3.0879x over previous
import functools

import jax
import jax.numpy as jnp
from jax import lax
from jax.experimental import pallas as pl
from jax.experimental.pallas import tpu as pltpu
from jax.experimental.pallas import tpu_sc as plsc

_N = 10000
_E = 320000
_D = 128

_NC = 2
_NS = 16
_NW = _NC * _NS
_C = 128
_CHUNKS = 80
_EPAD = _NW * _CHUNKS * _C
_NP = 10240
_RPT = _NP // _NS


def _matmul_silu(x, w):
    def body(x_ref, w_ref, y_ref):
        z = jnp.dot(x_ref[...], w_ref[...], preferred_element_type=jnp.float32)
        y_ref[...] = z * jax.nn.sigmoid(z)

    return pl.pallas_call(
        body,
        out_shape=jax.ShapeDtypeStruct((_N, _D), jnp.float32),
    )(x, w)


def _combine(x, agg_parts, deg_parts):
    def body(x_ref, agg_ref, deg_ref, o_ref):
        agg = agg_ref[0, 0:_N, :] + agg_ref[1, 0:_N, :]
        deg = deg_ref[0, 0:_N, 0:1] + deg_ref[1, 0:_N, 0:1]
        o_ref[...] = x_ref[...] + agg / jnp.maximum(deg, 1.0)

    return pl.pallas_call(
        body,
        out_shape=jax.ShapeDtypeStruct((_N, _D), jnp.float32),
    )(x, agg_parts, deg_parts)


def _make_sc_scatter():
    mesh = plsc.VectorSubcoreMesh(core_axis_name="c", subcore_axis_name="s")

    @functools.partial(
        pl.kernel,
        mesh=mesh,
        out_type=(
            jax.ShapeDtypeStruct((_NC * _NP, _D), jnp.float32),
            jax.ShapeDtypeStruct((_NC * _NP, _D), jnp.float32),
        ),
        scratch_types=[
            pltpu.VMEM((_C,), jnp.int32),
            pltpu.VMEM((_C,), jnp.int32),
            pltpu.VMEM((_C,), jnp.int32),
            pltpu.VMEM((_C, _D), jnp.float32),
            pltpu.VMEM((_C, _D), jnp.float32),
            pltpu.VMEM_SHARED((_NP, _D), jnp.float32),
            pltpu.SemaphoreType.DMA,
        ],
    )
    def sc_scatter(y_hbm, src_hbm, dst_hbm, zrow_hbm, ones_hbm, rid_hbm,
                   agg_out, deg_out,
                   src_v, dst_v, rid_v, rows_v, cst_v, agg_sh, sem):
        cid = lax.axis_index("c")
        sid = lax.axis_index("s")
        wid = sid * _NC + cid
        r0 = sid * _RPT
        base = wid * _CHUNKS * _C

        pltpu.sync_copy(zrow_hbm, cst_v)

        def zinit(i, carry):
            pltpu.sync_copy(rid_hbm.at[pl.ds(r0 + i * _C, _C)], rid_v)
            pltpu.sync_copy(cst_v, agg_sh.at[rid_v])
            return carry

        lax.fori_loop(0, _RPT // _C, zinit, 0)
        plsc.subcore_barrier()

        def chunk1(j, carry):
            off = base + j * _C
            pltpu.sync_copy(src_hbm.at[pl.ds(off, _C)], src_v)
            pltpu.sync_copy(dst_hbm.at[pl.ds(off, _C)], dst_v)
            pltpu.async_copy(y_hbm.at[src_v], rows_v, sem).wait()
            pltpu.sync_copy(rows_v, agg_sh.at[dst_v], add=True)
            return carry

        lax.fori_loop(0, _CHUNKS, chunk1, 0)
        plsc.subcore_barrier()

        def wback1(i, carry):
            ro = r0 + i * _C
            pltpu.sync_copy(rid_hbm.at[pl.ds(ro, _C)], rid_v)
            pltpu.async_copy(agg_sh.at[rid_v], rows_v, sem).wait()
            pltpu.sync_copy(rows_v, agg_out.at[pl.ds(cid * _NP + ro, _C)])
            return carry

        lax.fori_loop(0, _RPT // _C, wback1, 0)
        plsc.subcore_barrier()

        def zinit2(i, carry):
            pltpu.sync_copy(rid_hbm.at[pl.ds(r0 + i * _C, _C)], rid_v)
            pltpu.sync_copy(cst_v, agg_sh.at[rid_v])
            return carry

        lax.fori_loop(0, _RPT // _C, zinit2, 0)
        pltpu.sync_copy(ones_hbm, cst_v)
        plsc.subcore_barrier()

        def chunk2(j, carry):
            off = base + j * _C
            pltpu.sync_copy(dst_hbm.at[pl.ds(off, _C)], dst_v)
            pltpu.sync_copy(cst_v, agg_sh.at[dst_v], add=True)
            return carry

        lax.fori_loop(0, _CHUNKS, chunk2, 0)
        plsc.subcore_barrier()

        def wback2(i, carry):
            ro = r0 + i * _C
            pltpu.sync_copy(rid_hbm.at[pl.ds(ro, _C)], rid_v)
            pltpu.async_copy(agg_sh.at[rid_v], rows_v, sem).wait()
            pltpu.sync_copy(rows_v, deg_out.at[pl.ds(cid * _NP + ro, _C)])
            return carry

        lax.fori_loop(0, _RPT // _C, wback2, 0)

    return sc_scatter


_sc_scatter = _make_sc_scatter()


def kernel(x, edge_index, W):
    assert x.shape == (_N, _D) and edge_index.shape == (2, _E)
    y = _matmul_silu(x, W)
    pad = _EPAD - _E
    src = jnp.concatenate([edge_index[0], jnp.zeros((pad,), jnp.int32)])
    dst = jnp.concatenate([edge_index[1], jnp.full((pad,), _N, jnp.int32)])
    zrow = jnp.zeros((_C, _D), jnp.float32)
    ones = jnp.ones((_C, _D), jnp.float32)
    rids = jnp.arange(_NP, dtype=jnp.int32)
    agg_parts, deg_parts = _sc_scatter(y, src, dst, zrow, ones, rids)
    return _combine(x, agg_parts.reshape(_NC, _NP, _D),
                    deg_parts.reshape(_NC, _NP, _D))


# speedup vs baseline: 3.6185x; 1.1718x over previous
import functools

import jax
import jax.numpy as jnp
from jax import lax
from jax.experimental import pallas as pl
from jax.experimental.pallas import tpu as pltpu
from jax.experimental.pallas import tpu_sc as plsc

_N = 10000
_E = 320000
_D = 128

_NC = 2
_NS = 16
_NW = _NC * _NS
_C = 128
_CHUNKS = 80
_HC = 40
_EPAD = _NW * _CHUNKS * _C
_NP = 10240
_RPT = _NP // _NS


def _matmul_silu(x, w):
    def body(x_ref, w_ref, y_ref):
        z = jnp.dot(x_ref[...], w_ref[...], preferred_element_type=jnp.float32)
        y_ref[...] = z * jax.nn.sigmoid(z)

    return pl.pallas_call(
        body,
        out_shape=jax.ShapeDtypeStruct((_N, _D), jnp.float32),
    )(x, w)


def _combine(x, agg_parts, deg_parts):
    def body(x_ref, agg_ref, deg_ref, o_ref):
        agg = agg_ref[0, 0:_N, :] + agg_ref[1, 0:_N, :]
        deg = deg_ref[0, 0:_N, 0:1] + deg_ref[1, 0:_N, 0:1]
        o_ref[...] = x_ref[...] + agg / jnp.maximum(deg, 1.0)

    return pl.pallas_call(
        body,
        out_shape=jax.ShapeDtypeStruct((_N, _D), jnp.float32),
    )(x, agg_parts, deg_parts)


def _make_sc_scatter():
    mesh = plsc.VectorSubcoreMesh(core_axis_name="c", subcore_axis_name="s")

    @functools.partial(
        pl.kernel,
        mesh=mesh,
        out_type=(
            jax.ShapeDtypeStruct((_NC * _NP, _D), jnp.float32),
            jax.ShapeDtypeStruct((_NC * _NP, _D), jnp.float32),
        ),
        scratch_types=[
            pltpu.VMEM((_HC, _C), jnp.int32),
            pltpu.VMEM((_HC, _C), jnp.int32),
            pltpu.VMEM((_C,), jnp.int32),
            pltpu.VMEM((_C, _D), jnp.float32),
            pltpu.VMEM((_C, _D), jnp.float32),
            pltpu.VMEM_SHARED((_NP, _D), jnp.float32),
            pltpu.SemaphoreType.DMA,
            pltpu.SemaphoreType.DMA,
            pltpu.SemaphoreType.DMA,
            pltpu.SemaphoreType.DMA,
        ],
    )
    def sc_scatter(y_hbm, src_hbm, dst_hbm, zrow_hbm, ones_hbm, rid_hbm,
                   agg_out, deg_out,
                   src_v, dst_v, rid_v, rows_a, rows_b, agg_sh,
                   gsem_a, gsem_b, ssem_a, ssem_b):
        cid = lax.axis_index("c")
        sid = lax.axis_index("s")
        wid = sid * _NC + cid
        r0 = sid * _RPT

        pltpu.sync_copy(zrow_hbm, rows_a)

        def zinit(i, carry):
            pltpu.sync_copy(rid_hbm.at[pl.ds(r0 + i * _C, _C)], rid_v)
            pltpu.sync_copy(rows_a, agg_sh.at[rid_v])
            return carry

        lax.fori_loop(0, _RPT // _C, zinit, 0)
        plsc.subcore_barrier()

        for h in range(_CHUNKS // _HC):
            pltpu.sync_copy(src_hbm.at[wid, pl.ds(h * _HC, _HC)], src_v)
            pltpu.sync_copy(dst_hbm.at[wid, pl.ds(h * _HC, _HC)], dst_v)
            pltpu.async_copy(y_hbm.at[src_v.at[0]], rows_a, gsem_a)
            pltpu.async_copy(y_hbm.at[src_v.at[1]], rows_b, gsem_b)

            def chunk1(i, carry):
                j = 2 * i
                ja = jnp.minimum(j + 2, _HC - 1)
                jb = jnp.minimum(j + 3, _HC - 1)
                pltpu.make_async_copy(
                    y_hbm.at[src_v.at[j]], rows_a, gsem_a).wait()
                pltpu.async_copy(rows_a, agg_sh.at[dst_v.at[j]], ssem_a,
                                 add=True)
                pltpu.make_async_copy(
                    y_hbm.at[src_v.at[j + 1]], rows_b, gsem_b).wait()
                pltpu.async_copy(rows_b, agg_sh.at[dst_v.at[j + 1]], ssem_b,
                                 add=True)
                pltpu.make_async_copy(
                    rows_a, agg_sh.at[dst_v.at[j]], ssem_a).wait()
                pltpu.async_copy(y_hbm.at[src_v.at[ja]], rows_a, gsem_a)
                pltpu.make_async_copy(
                    rows_b, agg_sh.at[dst_v.at[j + 1]], ssem_b).wait()
                pltpu.async_copy(y_hbm.at[src_v.at[jb]], rows_b, gsem_b)
                return carry

            lax.fori_loop(0, _HC // 2, chunk1, 0)
            pltpu.make_async_copy(y_hbm.at[src_v.at[0]], rows_a, gsem_a).wait()
            pltpu.make_async_copy(y_hbm.at[src_v.at[0]], rows_b, gsem_b).wait()
        plsc.subcore_barrier()

        def wback1(i, carry):
            ro = r0 + i * _C
            pltpu.sync_copy(rid_hbm.at[pl.ds(ro, _C)], rid_v)
            pltpu.async_copy(agg_sh.at[rid_v], rows_a, gsem_a).wait()
            pltpu.sync_copy(rows_a, agg_out.at[pl.ds(cid * _NP + ro, _C)])
            return carry

        lax.fori_loop(0, _RPT // _C, wback1, 0)
        plsc.subcore_barrier()

        pltpu.sync_copy(zrow_hbm, rows_b)

        def zinit2(i, carry):
            pltpu.sync_copy(rid_hbm.at[pl.ds(r0 + i * _C, _C)], rid_v)
            pltpu.sync_copy(rows_b, agg_sh.at[rid_v])
            return carry

        lax.fori_loop(0, _RPT // _C, zinit2, 0)
        pltpu.sync_copy(ones_hbm, rows_a)
        plsc.subcore_barrier()

        for h in range(_CHUNKS // _HC):
            pltpu.sync_copy(dst_hbm.at[wid, pl.ds(h * _HC, _HC)], dst_v)

            def chunk2(j, carry):
                pltpu.async_copy(rows_a, agg_sh.at[dst_v.at[j]], ssem_a,
                                 add=True)
                return carry

            lax.fori_loop(0, _HC, chunk2, 0)

            def chunk2w(j, carry):
                pltpu.make_async_copy(
                    rows_a, agg_sh.at[dst_v.at[0]], ssem_a).wait()
                return carry

            lax.fori_loop(0, _HC, chunk2w, 0)
        plsc.subcore_barrier()

        def wback2(i, carry):
            ro = r0 + i * _C
            pltpu.sync_copy(rid_hbm.at[pl.ds(ro, _C)], rid_v)
            pltpu.async_copy(agg_sh.at[rid_v], rows_b, gsem_a).wait()
            pltpu.sync_copy(rows_b, deg_out.at[pl.ds(cid * _NP + ro, _C)])
            return carry

        lax.fori_loop(0, _RPT // _C, wback2, 0)

    return sc_scatter


_sc_scatter = _make_sc_scatter()


def kernel(x, edge_index, W):
    assert x.shape == (_N, _D) and edge_index.shape == (2, _E)
    y = _matmul_silu(x, W)
    pad = _EPAD - _E
    src = jnp.concatenate(
        [edge_index[0], jnp.zeros((pad,), jnp.int32)]).reshape(
            _NW, _CHUNKS, _C)
    dst = jnp.concatenate(
        [edge_index[1], jnp.full((pad,), _N, jnp.int32)]).reshape(
            _NW, _CHUNKS, _C)
    zrow = jnp.zeros((_C, _D), jnp.float32)
    ones = jnp.ones((_C, _D), jnp.float32)
    rids = jnp.arange(_NP, dtype=jnp.int32)
    agg_parts, deg_parts = _sc_scatter(y, src, dst, zrow, ones, rids)
    return _combine(x, agg_parts.reshape(_NC, _NP, _D),
                    deg_parts.reshape(_NC, _NP, _D))
